# baseline (speedup 1.0000x reference)
import functools

import jax
import jax.numpy as jnp
from jax import lax
from jax.experimental import pallas as pl
from jax.experimental.pallas import tpu as pltpu
from jax.experimental.pallas import tpu_sc as plsc

H = 4
O = 32
LEAK = 0.2

_SC_CORES = 2
_SC_SUBCORES = 16
_NW = _SC_CORES * _SC_SUBCORES


def _sc_gather(table, idx):
    V, D = table.shape
    B = idx.shape[0]
    bpw = B // _NW
    mesh = plsc.VectorSubcoreMesh(core_axis_name="c", subcore_axis_name="s")

    @functools.partial(
        pl.kernel,
        mesh=mesh,
        out_type=jax.ShapeDtypeStruct((B, D), jnp.float32),
        scratch_types=[
            pltpu.VMEM((bpw,), jnp.int32),
            pltpu.VMEM((bpw, D), jnp.float32),
            pltpu.SemaphoreType.DMA,
        ],
    )
    def gk(table_hbm, idx_hbm, out_hbm, idx_v, rows_v, sem):
        wid = lax.axis_index("s") * _SC_CORES + lax.axis_index("c")
        base = wid * bpw
        pltpu.sync_copy(idx_hbm.at[pl.ds(base, bpw)], idx_v)
        pltpu.async_copy(table_hbm.at[idx_v], rows_v, sem).wait()
        pltpu.sync_copy(rows_v, out_hbm.at[pl.ds(base, bpw)])

    return gk(table, idx)


def _gat_mega(adj, h0, Waug_all, Wsmall_all):
    Ntot = adj.shape[0]
    Lm = Waug_all.shape[0]
    BI = 512
    NI = Ntot // BI
    PH = 1 + NI
    CA = 64 * H

    def body(adj_ref, h0_ref, waug_ref, wsmall_ref, out_ref,
             m8_s, ha_s, hb_s, wh_s, f_s, fdt_s, maxd_s):
        s = pl.program_id(0)
        h_ins = [h0_ref, ha_s, hb_s]
        h_outs = [ha_s, hb_s, None]
        for l in range(Lm):
            base = l * PH

            @pl.when(s == base)
            def _proj_phase(l=l):
                hcur = h_ins[l][...]
                wh = jnp.dot(hcur, waug_ref[l],
                             preferred_element_type=jnp.float32)
                col = lax.broadcasted_iota(jnp.int32, (Ntot, CA), 1)
                wh_s[...] = jnp.where(col % 64 == O, 1.0, wh
                                      ).astype(jnp.bfloat16)
                f = jnp.dot(hcur, wsmall_ref[l],
                            preferred_element_type=jnp.float32)
                f_s[...] = f
                fdt_s[...] = f.T
                maxd_s[...] = jnp.broadcast_to(
                    jnp.max(f, axis=0, keepdims=True), (8, 16))

            @pl.when((s > base) & (s < base + PH))
            def _attn_phase(l=l):
                ph = s - base - 1
                r0 = ph * BI
                if l == 0:
                    rows = r0 + lax.broadcasted_iota(
                        jnp.int32, (BI, Ntot), 0)
                    cols = lax.broadcasted_iota(jnp.int32, (BI, Ntot), 1)
                    keep_f = jnp.where(
                        (adj_ref[...] > 0) | (rows == cols), 1.0, 0.0)
                    m8_s[pl.ds(r0, BI), :] = keep_f.astype(jnp.int8)
                    keep = keep_f.astype(jnp.bfloat16)
                else:
                    keep = m8_s[pl.ds(r0, BI), :].astype(jnp.bfloat16)
                fi = f_s[pl.ds(r0, BI), :]
                tot = None
                for hh in range(H):
                    md = maxd_s[0, 8 + hh]
                    t = fi[:, hh:hh + 1] + md
                    m = jnp.maximum(t, LEAK * t)
                    a1 = jnp.exp(t - m).astype(jnp.bfloat16)
                    a2 = jnp.exp(LEAK * t - m).astype(jnp.bfloat16)
                    u = fdt_s[8 + hh:9 + hh, :] - md
                    b1 = jnp.exp(u).astype(jnp.bfloat16)
                    b2 = jnp.exp(LEAK * u).astype(jnp.bfloat16)
                    p = jnp.maximum(a1 * b1, a2 * b2) * keep
                    r = jnp.dot(p, wh_s[:, 64 * hh:64 * (hh + 1)],
                                preferred_element_type=jnp.float32)
                    o = r[:, 0:O] / r[:, O:O + 1]
                    if l < Lm - 1:
                        o = o + h_ins[l][pl.ds(r0, BI), O * hh:O * (hh + 1)]
                        h_outs[l][pl.ds(r0, BI), O * hh:O * (hh + 1)] = (
                            jnp.where(o > 0, o, jnp.exp(o) - 1.0))
                    else:
                        tot = o if tot is None else tot + o
                if l == Lm - 1:
                    out_ref[...] = tot * (1.0 / H)

    last_base = (Lm - 1) * PH

    return pl.pallas_call(
        body,
        grid=(Lm * PH,),
        in_specs=[
            pl.BlockSpec((BI, Ntot),
                         lambda s: (jnp.clip(s - 1, 0, NI - 1), 0)),
            pl.BlockSpec((Ntot, H * O), lambda s: (0, 0)),
            pl.BlockSpec((Lm, H * O, CA), lambda s: (0, 0, 0)),
            pl.BlockSpec((Lm, H * O, 16), lambda s: (0, 0, 0)),
        ],
        out_specs=[
            pl.BlockSpec(
                (BI, O),
                lambda s: (jnp.clip(s - last_base - 1, 0, NI - 1), 0)),
        ],
        out_shape=[jax.ShapeDtypeStruct((Ntot, O), jnp.float32)],
        scratch_shapes=[
            pltpu.VMEM((Ntot, Ntot), jnp.int8),
            pltpu.VMEM((Ntot, H * O), jnp.float32),
            pltpu.VMEM((Ntot, H * O), jnp.float32),
            pltpu.VMEM((Ntot, CA), jnp.bfloat16),
            pltpu.VMEM((Ntot, 16), jnp.float32),
            pltpu.VMEM((16, Ntot), jnp.float32),
            pltpu.VMEM((8, 16), jnp.float32),
        ],
        compiler_params=pltpu.CompilerParams(
            dimension_semantics=("arbitrary",)),
    )(adj, h0, Waug_all, Wsmall_all)[0]


def kernel(tcword_id, adj, emb, W, a_src, a_dst):
    L = W.shape[0]
    V, D = emb.shape
    idx = tcword_id.astype(jnp.int32)
    h = jnp.take(emb, idx, axis=0)

    Wt = jnp.transpose(W, (0, 2, 1, 3))
    Waug_all = jnp.pad(Wt, ((0, 0), (0, 0), (0, 0), (0, 64 - O))
                       ).reshape(L, D, 64 * H)
    Wfl_all = Wt.reshape(L, D, H * O)
    ind = jnp.kron(jnp.eye(H, dtype=jnp.float32), jnp.ones((O, 1), jnp.float32))
    zL = jnp.zeros((L, H * O, H), jnp.float32)
    Asrc_all = ind[None] * a_src.reshape(L, H * O, 1)
    Adst_all = ind[None] * a_dst.reshape(L, H * O, 1)
    Apad_all = jnp.concatenate([Asrc_all, zL, Adst_all, zL], axis=2)
    Wsmall_all = jnp.einsum('lde,lef->ldf', Wfl_all, Apad_all)

    return _gat_mega(adj, h, Waug_all, Wsmall_all)


# speedup vs baseline: 1.0646x; 1.0646x over previous
import functools

import jax
import jax.numpy as jnp
from jax import lax
from jax.experimental import pallas as pl
from jax.experimental.pallas import tpu as pltpu
from jax.experimental.pallas import tpu_sc as plsc

H = 4
O = 32
LEAK = 0.2

_SC_CORES = 2
_SC_SUBCORES = 16
_NW = _SC_CORES * _SC_SUBCORES


def _sc_gather(table, idx):
    V, D = table.shape
    B = idx.shape[0]
    bpw = B // _NW
    mesh = plsc.VectorSubcoreMesh(core_axis_name="c", subcore_axis_name="s")

    @functools.partial(
        pl.kernel,
        mesh=mesh,
        out_type=jax.ShapeDtypeStruct((B, D), jnp.float32),
        scratch_types=[
            pltpu.VMEM((bpw,), jnp.int32),
            pltpu.VMEM((bpw, D), jnp.float32),
            pltpu.SemaphoreType.DMA,
        ],
    )
    def gk(table_hbm, idx_hbm, out_hbm, idx_v, rows_v, sem):
        wid = lax.axis_index("s") * _SC_CORES + lax.axis_index("c")
        base = wid * bpw
        pltpu.sync_copy(idx_hbm.at[pl.ds(base, bpw)], idx_v)
        pltpu.async_copy(table_hbm.at[idx_v], rows_v, sem).wait()
        pltpu.sync_copy(rows_v, out_hbm.at[pl.ds(base, bpw)])

    return gk(table, idx)


def _gat_mega(adj, h0, Waug_all, Wsmall_all):
    Ntot = adj.shape[0]
    Lm = Waug_all.shape[0]
    BI0 = 512
    NI0 = Ntot // BI0
    PH0 = 1 + NI0
    BI1 = 1024
    NI1 = Ntot // BI1
    PH1 = 1 + NI1
    CA = 64 * H

    def body(adj_ref, h0_ref, waug_ref, wsmall_ref, out_ref,
             m8_s, ha_s, hb_s, wh_s, f_s, fdt_s, maxd_s):
        s = pl.program_id(0)
        h_ins = [h0_ref, ha_s, hb_s]
        h_outs = [ha_s, hb_s, None]
        bases = [0, PH0, PH0 + PH1]
        bis = [BI0, BI1, BI1]
        for l in range(Lm):
            base = bases[l]
            BI = bis[l]
            NIl = Ntot // BI

            @pl.when(s == base)
            def _proj_phase(l=l):
                hcur = h_ins[l][...]
                wh = jnp.dot(hcur, waug_ref[l],
                             preferred_element_type=jnp.float32)
                col = lax.broadcasted_iota(jnp.int32, (Ntot, CA), 1)
                wh_s[...] = jnp.where(col % 64 == O, 1.0, wh
                                      ).astype(jnp.bfloat16)
                f = jnp.dot(hcur, wsmall_ref[l],
                            preferred_element_type=jnp.float32)
                f_s[...] = f
                fdt_s[...] = f.T
                maxd_s[...] = jnp.broadcast_to(
                    jnp.max(f, axis=0, keepdims=True), (8, 16))

            @pl.when((s > base) & (s < base + 1 + NIl))
            def _attn_phase(l=l, base=base, BI=BI):
                ph = s - base - 1
                r0 = ph * BI
                if l == 0:
                    rows = r0 + lax.broadcasted_iota(
                        jnp.int32, (BI, Ntot), 0)
                    cols = lax.broadcasted_iota(jnp.int32, (BI, Ntot), 1)
                    keep_f = jnp.where(
                        (adj_ref[...] > 0) | (rows == cols), 1.0, 0.0)
                    m8_s[pl.ds(r0, BI), :] = keep_f.astype(jnp.int8)
                    keep = keep_f.astype(jnp.bfloat16)
                else:
                    keep = m8_s[pl.ds(r0, BI), :].astype(jnp.bfloat16)
                fi = f_s[pl.ds(r0, BI), :]
                tot = None
                for hh in range(H):
                    md = maxd_s[0, 8 + hh]
                    t = fi[:, hh:hh + 1] + md
                    m = jnp.maximum(t, LEAK * t)
                    a1 = jnp.exp(t - m).astype(jnp.bfloat16)
                    a2 = jnp.exp(LEAK * t - m).astype(jnp.bfloat16)
                    u = fdt_s[8 + hh:9 + hh, :] - md
                    b1 = jnp.exp(u).astype(jnp.bfloat16)
                    b2 = jnp.exp(LEAK * u).astype(jnp.bfloat16)
                    p = jnp.maximum(a1 * b1, a2 * b2) * keep
                    r = jnp.dot(p, wh_s[:, 64 * hh:64 * (hh + 1)],
                                preferred_element_type=jnp.float32)
                    o = r[:, 0:O] / r[:, O:O + 1]
                    if l < Lm - 1:
                        o = o + h_ins[l][pl.ds(r0, BI), O * hh:O * (hh + 1)]
                        h_outs[l][pl.ds(r0, BI), O * hh:O * (hh + 1)] = (
                            jnp.where(o > 0, o, jnp.exp(o) - 1.0))
                    else:
                        tot = o if tot is None else tot + o
                if l == Lm - 1:
                    out_ref[...] = tot * (1.0 / H)

    last_base = PH0 + PH1

    return pl.pallas_call(
        body,
        grid=(PH0 + 2 * PH1,),
        in_specs=[
            pl.BlockSpec((BI0, Ntot),
                         lambda s: (jnp.clip(s - 1, 0, NI0 - 1), 0)),
            pl.BlockSpec((Ntot, H * O), lambda s: (0, 0)),
            pl.BlockSpec((Lm, H * O, CA), lambda s: (0, 0, 0)),
            pl.BlockSpec((Lm, H * O, 16), lambda s: (0, 0, 0)),
        ],
        out_specs=[
            pl.BlockSpec(
                (BI1, O),
                lambda s: (jnp.clip(s - last_base - 1, 0, NI1 - 1), 0)),
        ],
        out_shape=[jax.ShapeDtypeStruct((Ntot, O), jnp.float32)],
        scratch_shapes=[
            pltpu.VMEM((Ntot, Ntot), jnp.int8),
            pltpu.VMEM((Ntot, H * O), jnp.float32),
            pltpu.VMEM((Ntot, H * O), jnp.float32),
            pltpu.VMEM((Ntot, CA), jnp.bfloat16),
            pltpu.VMEM((Ntot, 16), jnp.float32),
            pltpu.VMEM((16, Ntot), jnp.float32),
            pltpu.VMEM((8, 16), jnp.float32),
        ],
        compiler_params=pltpu.CompilerParams(
            dimension_semantics=("arbitrary",)),
    )(adj, h0, Waug_all, Wsmall_all)[0]


def kernel(tcword_id, adj, emb, W, a_src, a_dst):
    L = W.shape[0]
    V, D = emb.shape
    idx = tcword_id.astype(jnp.int32)
    h = _sc_gather(emb, idx)

    Wt = jnp.transpose(W, (0, 2, 1, 3))
    Waug_all = jnp.pad(Wt, ((0, 0), (0, 0), (0, 0), (0, 64 - O))
                       ).reshape(L, D, 64 * H)
    Wfl_all = Wt.reshape(L, D, H * O)
    ind = jnp.kron(jnp.eye(H, dtype=jnp.float32), jnp.ones((O, 1), jnp.float32))
    zL = jnp.zeros((L, H * O, H), jnp.float32)
    Asrc_all = ind[None] * a_src.reshape(L, H * O, 1)
    Adst_all = ind[None] * a_dst.reshape(L, H * O, 1)
    Apad_all = jnp.concatenate([Asrc_all, zL, Adst_all, zL], axis=2)
    Wsmall_all = jnp.einsum('lde,lef->ldf', Wfl_all, Apad_all)

    return _gat_mega(adj, h, Waug_all, Wsmall_all)
